# Initial kernel scaffold; baseline (speedup 1.0000x reference)
import jax
import jax.numpy as jnp
from jax import lax
from jax.experimental import pallas as pl
from jax.experimental.pallas import tpu as pltpu
from jax.experimental.pallas import tpu_sc as plsc

NC = 2
NS = 16
L = 16
NW = NC * NS

R = 1024
C26 = 26
B = R * C26
D = 1000
BPW = B // NW
CHUNK = 64
NCHUNK = BPW // CHUNK


def _onehot_body(xs_hbm, out_hbm, idx_v, buf):
    cid = lax.axis_index("c")
    sid = lax.axis_index("s")
    wid = sid * NC + cid
    base_row = wid * BPW

    pltpu.sync_copy(xs_hbm.at[pl.ds(base_row, BPW)], idx_v)

    zeros16 = jnp.zeros((L,), jnp.float32)
    ones16 = jnp.ones((L,), jnp.float32)
    lane = lax.broadcasted_iota(jnp.int32, (L,), 0)

    @pl.loop(0, CHUNK * D, step=L)
    def _(i):
        buf.at[pl.ds(i, L)][...] = zeros16

    @pl.loop(0, NCHUNK)
    def _(c):
        @pl.loop(0, CHUNK, step=L)
        def _(r):
            cols = idx_v.at[pl.ds(c * CHUNK + r, L)][...]
            flat = (r + lane) * D + cols
            plsc.store_scatter(buf, [flat], ones16)

        pltpu.sync_copy(
            buf, out_hbm.at[pl.ds((base_row + c * CHUNK) * D, CHUNK * D)]
        )

        @pl.loop(0, CHUNK, step=L)
        def _(r):
            cols = idx_v.at[pl.ds(c * CHUNK + r, L)][...]
            flat = (r + lane) * D + cols
            plsc.store_scatter(buf, [flat], zeros16)


@jax.jit
def _onehot_expand(xs_flat):
    mesh = plsc.VectorSubcoreMesh(core_axis_name="c", subcore_axis_name="s")
    run = pl.kernel(
        _onehot_body,
        out_type=jax.ShapeDtypeStruct((B * D,), jnp.float32),
        mesh=mesh,
        scratch_types=[
            pltpu.VMEM((BPW,), jnp.int32),
            pltpu.VMEM((CHUNK * D,), jnp.float32),
        ],
    )
    return run(xs_flat)


def kernel(xs, weights):
    del weights
    xs_flat = xs.reshape(B).astype(jnp.int32)
    return _onehot_expand(xs_flat).reshape(R, C26, D)


# speedup vs baseline: 1.1377x; 1.1377x over previous
import dataclasses

import jax
import jax.numpy as jnp
from jax import lax
from jax.experimental import pallas as pl
from jax.experimental.pallas import tpu as pltpu
from jax.experimental.pallas import tpu_sc as plsc

NC = 2
NS = 16
L = 16
NW = NC * NS

R = 1024
C26 = 26
B = R * C26
D = 1000
BPW = B // NW
CHUNK = 64
NCHUNK = BPW // CHUNK


def _onehot_body(xs_hbm, out_hbm, idx_v, buf):
    cid = lax.axis_index("c")
    sid = lax.axis_index("s")
    wid = sid * NC + cid
    base_row = wid * BPW

    pltpu.sync_copy(xs_hbm.at[pl.ds(base_row, BPW)], idx_v)

    zeros16 = jnp.zeros((L,), jnp.float32)
    ones16 = jnp.ones((L,), jnp.float32)
    lane = lax.broadcasted_iota(jnp.int32, (L,), 0)

    @pl.loop(0, CHUNK * D, step=L)
    def _(i):
        buf.at[pl.ds(i, L)][...] = zeros16

    @pl.loop(0, NCHUNK)
    def _(c):
        @pl.loop(0, CHUNK, step=L)
        def _(r):
            cols = idx_v.at[pl.ds(c * CHUNK + r, L)][...]
            flat = (r + lane) * D + cols
            plsc.store_scatter(buf, [flat], ones16)

        pltpu.sync_copy(
            buf, out_hbm.at[pl.ds((base_row + c * CHUNK) * D, CHUNK * D)]
        )

        @pl.loop(0, CHUNK, step=L)
        def _(r):
            cols = idx_v.at[pl.ds(c * CHUNK + r, L)][...]
            flat = (r + lane) * D + cols
            plsc.store_scatter(buf, [flat], zeros16)


@jax.jit
def _onehot_expand(xs_flat):
    mesh = plsc.VectorSubcoreMesh(core_axis_name="c", subcore_axis_name="s")
    cp = pltpu.CompilerParams()
    if "needs_layout_passes" in pltpu.CompilerParams.__dataclass_fields__:
        cp = dataclasses.replace(cp, needs_layout_passes=False)
    run = pl.kernel(
        _onehot_body,
        out_type=jax.ShapeDtypeStruct((B * D,), jnp.float32),
        mesh=mesh,
        scratch_types=[
            pltpu.VMEM((BPW,), jnp.int32),
            pltpu.VMEM((CHUNK * D,), jnp.float32),
        ],
        compiler_params=cp,
    )
    return run(xs_flat)


def kernel(xs, weights):
    del weights
    xs_flat = xs.reshape(B).astype(jnp.int32)
    return _onehot_expand(xs_flat).reshape(R, C26, D)


# speedup vs baseline: 2.0102x; 1.7668x over previous
import dataclasses

import jax
import jax.numpy as jnp
import numpy as np
from jax import lax
from jax.experimental import pallas as pl
from jax.experimental.pallas import tpu as pltpu
from jax.experimental.pallas import tpu_sc as plsc

NC = 2
NS = 16
L = 16
NW = NC * NS

R = 1024
C26 = 26
B = R * C26
D = 1000
IPW = R // NW
KI = 2
NCHUNK = IPW // KI
CL = KI * C26
NG = (CL + L - 1) // L
PAT = NG * L
TCOLS = NCHUNK * PAT

_l = np.minimum(np.arange(PAT), CL - 1)
_IV = np.asarray(_l // C26, dtype=np.int32)
_JV = np.asarray(_l % C26, dtype=np.int32)


def _onehot_body(cols_hbm, zeros_hbm, iv_hbm, jv_hbm, out_hbm, cols_v, iv_v,
                 jv_v, buf):
    cid = lax.axis_index("c")
    sid = lax.axis_index("s")
    wid = sid * NC + cid
    base_i = wid * IPW

    pltpu.sync_copy(cols_hbm.at[pl.ds(wid * TCOLS, TCOLS)], cols_v)
    pltpu.sync_copy(iv_hbm, iv_v)
    pltpu.sync_copy(jv_hbm, jv_v)
    pltpu.sync_copy(zeros_hbm, buf)

    zeros16 = jnp.zeros((L,), jnp.float32)
    ones16 = jnp.ones((L,), jnp.float32)
    lane = lax.broadcasted_iota(jnp.int32, (L,), 0)
    tail_mask = lane < (CL - (NG - 1) * L)

    def scatter_chunk(c, val16):
        for g in range(NG):
            iv16 = iv_v.at[pl.ds(g * L, L)][...]
            jv16 = jv_v.at[pl.ds(g * L, L)][...]
            cols = cols_v.at[pl.ds(c * PAT + g * L, L)][...]
            mask = tail_mask if g == NG - 1 else None
            plsc.store_scatter(buf, [iv16, jv16, cols], val16, mask=mask)

    @pl.loop(0, NCHUNK)
    def _(c):
        scatter_chunk(c, ones16)
        pltpu.sync_copy(buf, out_hbm.at[pl.ds(base_i + c * KI, KI)])
        scatter_chunk(c, zeros16)


@jax.jit
def _onehot_expand(xs_flat):
    cols = xs_flat.reshape(R // KI, CL)
    cols = jnp.pad(cols, ((0, 0), (0, PAT - CL))).reshape(-1)

    mesh = plsc.VectorSubcoreMesh(core_axis_name="c", subcore_axis_name="s")
    cp = pltpu.CompilerParams()
    if "needs_layout_passes" in pltpu.CompilerParams.__dataclass_fields__:
        cp = dataclasses.replace(cp, needs_layout_passes=False)
    run = pl.kernel(
        _onehot_body,
        out_type=jax.ShapeDtypeStruct((R, C26, D), jnp.float32),
        mesh=mesh,
        scratch_types=[
            pltpu.VMEM((TCOLS,), jnp.int32),
            pltpu.VMEM((PAT,), jnp.int32),
            pltpu.VMEM((PAT,), jnp.int32),
            pltpu.VMEM((KI, C26, D), jnp.float32),
        ],
        compiler_params=cp,
    )
    return run(cols, jnp.zeros((KI, C26, D), jnp.float32), _IV, _JV)


def kernel(xs, weights):
    del weights
    xs_flat = xs.reshape(B).astype(jnp.int32)
    return _onehot_expand(xs_flat)


# speedup vs baseline: 2.0169x; 1.0033x over previous
import dataclasses

import jax
import jax.numpy as jnp
import numpy as np
from jax import lax
from jax.experimental import pallas as pl
from jax.experimental.pallas import tpu as pltpu
from jax.experimental.pallas import tpu_sc as plsc

NC = 2
NS = 16
L = 16
NW = NC * NS

R = 1024
C26 = 26
B = R * C26
D = 1000
IPW = R // NW
KI = 2
NCHUNK = IPW // KI
CL = KI * C26
NG = (CL + L - 1) // L
PAT = NG * L
TCOLS = NCHUNK * PAT

_l = np.minimum(np.arange(PAT), CL - 1)
_IV = np.asarray(_l // C26, dtype=np.int32)
_JV = np.asarray(_l % C26, dtype=np.int32)


def _onehot_body(cols_hbm, zeros_hbm, iv_hbm, jv_hbm, out_hbm, cols_v, iv_v,
                 jv_v, buf):
    cid = lax.axis_index("c")
    sid = lax.axis_index("s")
    wid = sid * NC + cid
    base_i = wid * IPW

    pltpu.sync_copy(cols_hbm.at[pl.ds(wid * TCOLS, TCOLS)], cols_v)
    pltpu.sync_copy(iv_hbm, iv_v)
    pltpu.sync_copy(jv_hbm, jv_v)
    pltpu.sync_copy(zeros_hbm, buf)

    zeros16 = jnp.zeros((L,), jnp.float32)
    ones16 = jnp.ones((L,), jnp.float32)
    lane = lax.broadcasted_iota(jnp.int32, (L,), 0)
    tail_mask = lane < (CL - (NG - 1) * L)

    def scatter_chunk(c, val16):
        for g in range(NG):
            iv16 = iv_v.at[pl.ds(g * L, L)][...]
            jv16 = jv_v.at[pl.ds(g * L, L)][...]
            cols = cols_v.at[pl.ds(c * PAT + g * L, L)][...]
            mask = tail_mask if g == NG - 1 else None
            plsc.store_scatter(buf, [iv16, jv16, cols], val16, mask=mask)

    @pl.loop(0, NCHUNK)
    def _(c):
        scatter_chunk(c, ones16)
        pltpu.sync_copy(buf, out_hbm.at[pl.ds(base_i + c * KI, KI)])
        scatter_chunk(c, zeros16)


@jax.jit
def _onehot_expand(xs_flat):
    cols = xs_flat.reshape(R // KI, CL)
    cols = jnp.pad(cols, ((0, 0), (0, PAT - CL))).reshape(-1)

    mesh = plsc.VectorSubcoreMesh(core_axis_name="c", subcore_axis_name="s")
    cp = pltpu.CompilerParams()
    if "needs_layout_passes" in pltpu.CompilerParams.__dataclass_fields__:
        cp = dataclasses.replace(cp, needs_layout_passes=False)
    if "use_tc_tiling_on_sc" in pltpu.CompilerParams.__dataclass_fields__:
        cp = dataclasses.replace(cp, use_tc_tiling_on_sc=True)
    run = pl.kernel(
        _onehot_body,
        out_type=jax.ShapeDtypeStruct((R, C26, D), jnp.float32),
        mesh=mesh,
        scratch_types=[
            pltpu.VMEM((TCOLS,), jnp.int32),
            pltpu.VMEM((PAT,), jnp.int32),
            pltpu.VMEM((PAT,), jnp.int32),
            pltpu.VMEM((KI, C26, D), jnp.float32),
        ],
        compiler_params=cp,
    )
    return run(cols, jnp.zeros((KI, C26, D), jnp.float32), _IV, _JV)


def kernel(xs, weights):
    del weights
    xs_flat = xs.reshape(B).astype(jnp.int32)
    return _onehot_expand(xs_flat)


# speedup vs baseline: 5.7919x; 2.8716x over previous
import dataclasses

import jax
import jax.numpy as jnp
from jax import lax
from jax.experimental import pallas as pl
from jax.experimental.pallas import tpu as pltpu
from jax.experimental.pallas import tpu_sc as plsc

NC = 2
NS = 16
L = 16
NW = NC * NS

R = 1024
C26 = 26
B = R * C26
D = 1000
LC = R // 128
NPAIR = C26 * LC
SLOTS = (NPAIR + NW - 1) // NW
KR = 200
NKR = D // KR


def _onehot_body(xst_hbm, zeros_hbm, out_hbm, cols_v, buf):
    cid = lax.axis_index("c")
    sid = lax.axis_index("s")
    wid = sid * NC + cid

    pltpu.sync_copy(zeros_hbm, buf)

    zeros16 = jnp.zeros((L,), jnp.float32)
    ones16 = jnp.ones((L,), jnp.float32)
    lane = lax.broadcasted_iota(jnp.int32, (L,), 0)

    @pl.loop(0, SLOTS)
    def _(s):
        p = wid + s * NW

        @pl.when(p < NPAIR)
        def _():
            j = p >> 3
            c = p & 7
            pltpu.sync_copy(xst_hbm.at[pl.ds(j * R + c * 128, 128)], cols_v)

            def scatter_range(k0, val16):
                for g in range(128 // L):
                    cols16 = cols_v.at[pl.ds(g * L, L)][...]
                    kl = cols16 - k0
                    mask = (cols16 >= k0) & (cols16 < k0 + KR)
                    il = g * L + lane
                    plsc.store_scatter(buf, [kl, il], val16, mask=mask)

            @pl.loop(0, NKR)
            def _(r):
                k0 = r * KR
                scatter_range(k0, ones16)
                pltpu.sync_copy(
                    buf,
                    out_hbm.at[j, pl.ds(k0, KR), pl.ds(c * 128, 128)],
                )
                scatter_range(k0, zeros16)


@jax.jit
def _onehot_expand(xs):
    xst = jnp.transpose(xs.astype(jnp.int32)).reshape(B)

    mesh = plsc.VectorSubcoreMesh(core_axis_name="c", subcore_axis_name="s")
    cp = pltpu.CompilerParams()
    if "needs_layout_passes" in pltpu.CompilerParams.__dataclass_fields__:
        cp = dataclasses.replace(cp, needs_layout_passes=False)
    run = pl.kernel(
        _onehot_body,
        out_type=jax.ShapeDtypeStruct((C26, D, R), jnp.float32),
        mesh=mesh,
        scratch_types=[
            pltpu.VMEM((128,), jnp.int32),
            pltpu.VMEM((KR, 128), jnp.float32),
        ],
        compiler_params=cp,
    )
    out_t = run(xst, jnp.zeros((KR, 128), jnp.float32))
    return jnp.transpose(out_t, (2, 0, 1))


def kernel(xs, weights):
    del weights
    return _onehot_expand(xs)
